# Initial kernel scaffold; baseline (speedup 1.0000x reference)
import jax
import jax.numpy as jnp
from jax.experimental import pallas as pl


def kernel(entity_emb_sr, entity_emb_tg, relation_emb_sr, relation_emb_tg, head_sr, tail_sr, relation_sr, head_tg, tail_tg, relation_tg):
    raise NotImplementedError("write your pallas kernel here")


# speedup vs baseline: 1.1870x; 1.1870x over previous
import functools
import math

import jax
import jax.numpy as jnp
from jax import lax
from jax.experimental import pallas as pl
from jax.experimental.pallas import tpu as pltpu
from jax.experimental.pallas import tpu_sc as plsc

N_ENT = 4096
N_REL = 512
N_TRI = 131072
DIM = 128
LANES = 16
NC = 2
NS = 16
NW = NC * NS
TRI_PER_W = N_TRI // NW
CHUNK = 128
N_CHUNK = TRI_PER_W // CHUNK
DENOM_INV = 1.0 / (3.0 * math.sqrt(DIM))

ROWS_PER_SC = N_ENT // NC
PASS_ROWS = 256
N_PASS = ROWS_PER_SC // PASS_ROWS
ACC = PASS_ROWS * N_ENT
TRI_PER_T = N_TRI // NS
STRIPE = ACC // NS
ZCHUNK = 16384

_mesh = plsc.VectorSubcoreMesh(core_axis_name="c", subcore_axis_name="s")


def _newton_sqrt(x):
    ib = lax.bitcast_convert_type(x, jnp.int32)
    ib = jnp.int32(0x5F3759DF) - lax.shift_right_arithmetic(ib, 1)
    y = lax.bitcast_convert_type(ib, jnp.float32)
    for _ in range(3):
        y = y * (1.5 - 0.5 * x * y * y)
    return x * y


def _score_body(ent_hbm, rel_hbm, h_hbm, t_hbm, r_hbm, scores_hbm,
                hidx, tidx, ridx, hrows, trows, rrows, scorebuf,
                sem0, sem1, sem2):
    cid = lax.axis_index("c")
    sid = lax.axis_index("s")
    wid = sid * NC + cid
    rowbase = wid * (TRI_PER_W // 128)
    pltpu.sync_copy(h_hbm.at[pl.ds(rowbase, N_CHUNK)], hidx)
    pltpu.sync_copy(t_hbm.at[pl.ds(rowbase, N_CHUNK)], tidx)
    pltpu.sync_copy(r_hbm.at[pl.ds(rowbase, N_CHUNK)], ridx)

    lane = lax.broadcasted_iota(jnp.int32, (LANES,), 0)
    perms = [jnp.mod(lane + s, LANES).reshape(LANES, 1) for s in (8, 4, 2, 1)]
    dnums = lax.GatherDimensionNumbers(
        offset_dims=(), collapsed_slice_dims=(0,), start_index_map=(0,))

    def _permute(x, perm):
        return lax.gather(x, perm, dnums, slice_sizes=(1,),
                          mode=lax.GatherScatterMode.PROMISE_IN_BOUNDS)

    @pl.loop(0, N_CHUNK)
    def _chunk(cidx):
        cp0 = pltpu.async_copy(ent_hbm.at[hidx.at[cidx]], hrows, sem0)
        cp1 = pltpu.async_copy(ent_hbm.at[tidx.at[cidx]], trows, sem1)
        cp2 = pltpu.async_copy(rel_hbm.at[ridx.at[cidx]], rrows, sem2)
        cp0.wait()
        cp1.wait()
        cp2.wait()

        @pl.loop(0, CHUNK // LANES)
        def _group(g):
            vec = jnp.zeros((LANES,), jnp.float32)
            for u in range(LANES):
                i = g * LANES + u
                acc = jnp.zeros((LANES,), jnp.float32)
                for j in range(DIM // LANES):
                    dh = hrows[i, pl.ds(j * LANES, LANES)]
                    dr = rrows[i, pl.ds(j * LANES, LANES)]
                    dt = trows[i, pl.ds(j * LANES, LANES)]
                    d = dh + dr - dt
                    acc = acc + d * d
                for perm in perms:
                    acc = acc + _permute(acc, perm)
                vec = jnp.where(lane == u, acc, vec)
            score = 1.0 - _newton_sqrt(vec) * DENOM_INV
            scorebuf[pl.ds(cidx * CHUNK + g * LANES, LANES)] = score

    pltpu.sync_copy(scorebuf, scores_hbm.at[pl.ds(wid * TRI_PER_W, TRI_PER_W)])


_score_call = pl.kernel(
    _score_body,
    out_type=jax.ShapeDtypeStruct((N_TRI,), jnp.float32),
    mesh=_mesh,
    scratch_types=[
        pltpu.VMEM((N_CHUNK, CHUNK), jnp.int32),
        pltpu.VMEM((N_CHUNK, CHUNK), jnp.int32),
        pltpu.VMEM((N_CHUNK, CHUNK), jnp.int32),
        pltpu.VMEM((CHUNK, DIM), jnp.float32),
        pltpu.VMEM((CHUNK, DIM), jnp.float32),
        pltpu.VMEM((CHUNK, DIM), jnp.float32),
        pltpu.VMEM((TRI_PER_W,), jnp.float32),
        pltpu.SemaphoreType.DMA,
        pltpu.SemaphoreType.DMA,
        pltpu.SemaphoreType.DMA,
    ],
)


def _scatter_body(h_hbm, t_hbm, s_hbm, out_hbm,
                  acc, hbuf, tbuf, sbuf, idx_stage, val_stage, zeros_v):
    cid = lax.axis_index("c")
    sid = lax.axis_index("s")
    tb = sid * TRI_PER_T
    pltpu.sync_copy(h_hbm.at[pl.ds(tb, TRI_PER_T)], hbuf)
    pltpu.sync_copy(t_hbm.at[pl.ds(tb, TRI_PER_T)], tbuf)
    pltpu.sync_copy(s_hbm.at[pl.ds(tb, TRI_PER_T)], sbuf)

    @pl.loop(0, ZCHUNK // LANES)
    def _zinit(k):
        zeros_v[pl.ds(k * LANES, LANES)] = jnp.zeros((LANES,), jnp.float32)

    for p in range(N_PASS):
        row_lo = cid * ROWS_PER_SC + p * PASS_ROWS
        for z in range(STRIPE // ZCHUNK):
            pltpu.sync_copy(zeros_v,
                            acc.at[pl.ds(sid * STRIPE + z * ZCHUNK, ZCHUNK)])
        plsc.subcore_barrier()

        @pl.loop(0, TRI_PER_T // CHUNK)
        def _rows(kr):
            off = kr * CHUNK
            for g in range(CHUNK // LANES):
                hv = hbuf[pl.ds(off + g * LANES, LANES)]
                tv = tbuf[pl.ds(off + g * LANES, LANES)]
                sv = sbuf[pl.ds(off + g * LANES, LANES)]
                loc = hv - row_lo
                m = (loc >= 0) & (loc < PASS_ROWS)
                lidx = jnp.where(m, loc * N_ENT + tv, 0)
                val = jnp.where(m, sv, 0.0)
                idx_stage[pl.ds(g * LANES, LANES)] = lidx
                val_stage[pl.ds(g * LANES, LANES)] = val
            pltpu.sync_copy(val_stage, acc.at[idx_stage], add=True)

        plsc.subcore_barrier()
        pl.delay(2000)
        out_off = row_lo * N_ENT + sid * STRIPE
        pltpu.sync_copy(acc.at[pl.ds(sid * STRIPE, STRIPE)],
                        out_hbm.at[pl.ds(out_off, STRIPE)])


_scatter_call = pl.kernel(
    _scatter_body,
    out_type=jax.ShapeDtypeStruct((N_ENT * N_ENT,), jnp.float32),
    mesh=_mesh,
    scratch_types=[
        pltpu.VMEM_SHARED((ACC,), jnp.float32),
        pltpu.VMEM((TRI_PER_T,), jnp.int32),
        pltpu.VMEM((TRI_PER_T,), jnp.int32),
        pltpu.VMEM((TRI_PER_T,), jnp.float32),
        pltpu.VMEM((CHUNK,), jnp.int32),
        pltpu.VMEM((CHUNK,), jnp.float32),
        pltpu.VMEM((ZCHUNK,), jnp.float32),
    ],
)


_DEBUG = 0


def _side(ent, rel, h, t, r):
    h = h.astype(jnp.int32)
    t = t.astype(jnp.int32)
    r = r.astype(jnp.int32)
    if _DEBUG != 2:
        scores = _score_call(ent, rel,
                             h.reshape(N_TRI // 128, 128),
                             t.reshape(N_TRI // 128, 128),
                             r.reshape(N_TRI // 128, 128))
    else:
        hv = jnp.take(ent, h, axis=0)
        tv = jnp.take(ent, t, axis=0)
        rv = jnp.take(rel, r, axis=0)
        scores = 1.0 - jnp.linalg.norm(hv + rv - tv, axis=1) * DENOM_INV
    if _DEBUG != 1:
        out = _scatter_call(h, t, scores)
        return out.reshape(N_ENT, N_ENT)
    return jnp.zeros((N_ENT, N_ENT), jnp.float32).at[h, t].add(scores)


def kernel(entity_emb_sr, entity_emb_tg, relation_emb_sr, relation_emb_tg,
           head_sr, tail_sr, relation_sr, head_tg, tail_tg, relation_tg):
    out_sr = _side(entity_emb_sr, relation_emb_sr, head_sr, tail_sr, relation_sr)
    out_tg = _side(entity_emb_tg, relation_emb_tg, head_tg, tail_tg, relation_tg)
    return (out_sr, out_tg)


# speedup vs baseline: 4.1992x; 3.5376x over previous
import functools
import math

import jax
import jax.numpy as jnp
from jax import lax
from jax.experimental import pallas as pl
from jax.experimental.pallas import tpu as pltpu
from jax.experimental.pallas import tpu_sc as plsc

N_ENT = 4096
N_REL = 512
N_TRI = 131072
DIM = 128
LANES = 16
NC = 2
NS = 16
NW = NC * NS
TRI_PER_W = N_TRI // NW
CHUNK = 128
N_CHUNK = TRI_PER_W // CHUNK
DENOM_INV = 1.0 / (3.0 * math.sqrt(DIM))

ROWS_PER_SC = N_ENT // NC
PASS_ROWS = 256
N_PASS = ROWS_PER_SC // PASS_ROWS
ACC = PASS_ROWS * N_ENT
TRI_PER_T = N_TRI // NS
STRIPE = ACC // NS
ZCHUNK = 16384

_mesh = plsc.VectorSubcoreMesh(core_axis_name="c", subcore_axis_name="s")


def _newton_sqrt(x):
    ib = lax.bitcast_convert_type(x, jnp.int32)
    ib = jnp.int32(0x5F3759DF) - lax.shift_right_arithmetic(ib, 1)
    y = lax.bitcast_convert_type(ib, jnp.float32)
    for _ in range(3):
        y = y * (1.5 - 0.5 * x * y * y)
    return x * y


def _score_body(ent_hbm, rel_hbm, h_hbm, t_hbm, r_hbm, scores_hbm,
                hidx, tidx, ridx, hrows0, trows0, rrows0,
                hrows1, trows1, rrows1, scorebuf, sem0, sem1):
    cid = lax.axis_index("c")
    sid = lax.axis_index("s")
    wid = sid * NC + cid
    rowbase = wid * (TRI_PER_W // 128)
    pltpu.sync_copy(h_hbm.at[pl.ds(rowbase, N_CHUNK)], hidx)
    pltpu.sync_copy(t_hbm.at[pl.ds(rowbase, N_CHUNK)], tidx)
    pltpu.sync_copy(r_hbm.at[pl.ds(rowbase, N_CHUNK)], ridx)

    lane = lax.broadcasted_iota(jnp.int32, (LANES,), 0)
    perms = [jnp.mod(lane + s, LANES).reshape(LANES, 1) for s in (8, 4, 2, 1)]
    dnums = lax.GatherDimensionNumbers(
        offset_dims=(), collapsed_slice_dims=(0,), start_index_map=(0,))

    def _permute(x, perm):
        return lax.gather(x, perm, dnums, slice_sizes=(1,),
                          mode=lax.GatherScatterMode.PROMISE_IN_BOUNDS)

    def _fire(cidx, hrows, trows, rrows, sem):
        pltpu.async_copy(ent_hbm.at[hidx.at[cidx]], hrows, sem)
        pltpu.async_copy(ent_hbm.at[tidx.at[cidx]], trows, sem)
        pltpu.async_copy(rel_hbm.at[ridx.at[cidx]], rrows, sem)

    def _drain(cidx, hrows, trows, rrows, sem):
        pltpu.make_async_copy(ent_hbm.at[hidx.at[cidx]], hrows, sem).wait()
        pltpu.make_async_copy(ent_hbm.at[tidx.at[cidx]], trows, sem).wait()
        pltpu.make_async_copy(rel_hbm.at[ridx.at[cidx]], rrows, sem).wait()

    def _compute(cidx, hrows, trows, rrows):
        @pl.loop(0, CHUNK // LANES)
        def _group(g):
            vec = jnp.zeros((LANES,), jnp.float32)
            for u in range(LANES):
                i = g * LANES + u
                acc = jnp.zeros((LANES,), jnp.float32)
                for j in range(DIM // LANES):
                    dh = hrows[i, pl.ds(j * LANES, LANES)]
                    dr = rrows[i, pl.ds(j * LANES, LANES)]
                    dt = trows[i, pl.ds(j * LANES, LANES)]
                    d = dh + dr - dt
                    acc = acc + d * d
                for perm in perms:
                    acc = acc + _permute(acc, perm)
                vec = jnp.where(lane == u, acc, vec)
            score = 1.0 - _newton_sqrt(vec) * DENOM_INV
            scorebuf[pl.ds(cidx * CHUNK + g * LANES, LANES)] = score

    _fire(0, hrows0, trows0, rrows0, sem0)

    @pl.loop(0, N_CHUNK // 2)
    def _chunk(k):
        c0 = 2 * k
        _fire(c0 + 1, hrows1, trows1, rrows1, sem1)
        _drain(c0, hrows0, trows0, rrows0, sem0)
        _compute(c0, hrows0, trows0, rrows0)

        @pl.when(k < N_CHUNK // 2 - 1)
        def _():
            _fire(c0 + 2, hrows0, trows0, rrows0, sem0)

        _drain(c0 + 1, hrows1, trows1, rrows1, sem1)
        _compute(c0 + 1, hrows1, trows1, rrows1)

    pltpu.sync_copy(scorebuf, scores_hbm.at[pl.ds(wid * TRI_PER_W, TRI_PER_W)])


_score_call = pl.kernel(
    _score_body,
    out_type=jax.ShapeDtypeStruct((N_TRI,), jnp.float32),
    mesh=_mesh,
    scratch_types=[
        pltpu.VMEM((N_CHUNK, CHUNK), jnp.int32),
        pltpu.VMEM((N_CHUNK, CHUNK), jnp.int32),
        pltpu.VMEM((N_CHUNK, CHUNK), jnp.int32),
        pltpu.VMEM((CHUNK, DIM), jnp.float32),
        pltpu.VMEM((CHUNK, DIM), jnp.float32),
        pltpu.VMEM((CHUNK, DIM), jnp.float32),
        pltpu.VMEM((CHUNK, DIM), jnp.float32),
        pltpu.VMEM((CHUNK, DIM), jnp.float32),
        pltpu.VMEM((CHUNK, DIM), jnp.float32),
        pltpu.VMEM((TRI_PER_W,), jnp.float32),
        pltpu.SemaphoreType.DMA,
        pltpu.SemaphoreType.DMA,
    ],
)


N_SCHUNK = TRI_PER_T // CHUNK


def _scatter_body(h_hbm, t_hbm, s_hbm, out_hbm,
                  acc, hbuf, gidx, vals, idxbuf, zeros_v, sem, zsem):
    cid = lax.axis_index("c")
    sid = lax.axis_index("s")
    tb = sid * (TRI_PER_T // CHUNK)
    cp0 = pltpu.async_copy(h_hbm.at[pl.ds(tb, N_SCHUNK)], hbuf, sem)
    cp1 = pltpu.async_copy(t_hbm.at[pl.ds(tb, N_SCHUNK)], gidx, zsem)
    cp2 = pltpu.async_copy(s_hbm.at[pl.ds(tb, N_SCHUNK)], vals, sem)
    cp0.wait()
    cp1.wait()
    cp2.wait()

    lane = lax.broadcasted_iota(jnp.int32, (LANES,), 0)
    dump = jnp.int32(ACC) + lane * 8

    @pl.loop(0, N_SCHUNK)
    def _pre(j):
        for g in range(CHUNK // LANES):
            hv = hbuf[j, pl.ds(g * LANES, LANES)]
            tv = gidx[j, pl.ds(g * LANES, LANES)]
            gidx[j, pl.ds(g * LANES, LANES)] = hv * N_ENT + tv

    @pl.loop(0, ZCHUNK // LANES)
    def _zinit(k):
        zeros_v[pl.ds(k * LANES, LANES)] = jnp.zeros((LANES,), jnp.float32)

    for p in range(N_PASS):
        base = (cid * ROWS_PER_SC + p * PASS_ROWS) * N_ENT

        zcps = [pltpu.async_copy(
                    zeros_v, acc.at[pl.ds(sid * STRIPE + z * ZCHUNK, ZCHUNK)],
                    zsem)
                for z in range(STRIPE // ZCHUNK)]

        @pl.loop(0, N_SCHUNK)
        def _idx(j):
            for g in range(CHUNK // LANES):
                gv = gidx[j, pl.ds(g * LANES, LANES)] - base
                m = (gv >= 0) & (gv < ACC)
                idxbuf[j, pl.ds(g * LANES, LANES)] = jnp.where(m, gv, dump)

        for cp in zcps:
            cp.wait()
        plsc.subcore_barrier()

        cps = [pltpu.async_copy(vals.at[j], acc.at[idxbuf.at[j]], sem,
                                add=True)
               for j in range(N_SCHUNK)]
        for cp in cps:
            cp.wait()

        plsc.subcore_barrier()
        pl.delay(2000)
        pltpu.sync_copy(acc.at[pl.ds(sid * STRIPE, STRIPE)],
                        out_hbm.at[pl.ds(base + sid * STRIPE, STRIPE)])


_scatter_call = pl.kernel(
    _scatter_body,
    out_type=jax.ShapeDtypeStruct((N_ENT * N_ENT,), jnp.float32),
    mesh=_mesh,
    scratch_types=[
        pltpu.VMEM_SHARED((ACC + 128,), jnp.float32),
        pltpu.VMEM((N_SCHUNK, CHUNK), jnp.int32),
        pltpu.VMEM((N_SCHUNK, CHUNK), jnp.int32),
        pltpu.VMEM((N_SCHUNK, CHUNK), jnp.float32),
        pltpu.VMEM((N_SCHUNK, CHUNK), jnp.int32),
        pltpu.VMEM((ZCHUNK,), jnp.float32),
        pltpu.SemaphoreType.DMA,
        pltpu.SemaphoreType.DMA,
    ],
)


_DEBUG = 0


def _side(ent, rel, h, t, r):
    h = h.astype(jnp.int32)
    t = t.astype(jnp.int32)
    r = r.astype(jnp.int32)
    h2 = h.reshape(N_TRI // 128, 128)
    t2 = t.reshape(N_TRI // 128, 128)
    if _DEBUG != 2:
        scores = _score_call(ent, rel, h2, t2, r.reshape(N_TRI // 128, 128))
    else:
        hv = jnp.take(ent, h, axis=0)
        tv = jnp.take(ent, t, axis=0)
        rv = jnp.take(rel, r, axis=0)
        scores = 1.0 - jnp.linalg.norm(hv + rv - tv, axis=1) * DENOM_INV
    if _DEBUG != 1:
        out = _scatter_call(h2, t2, scores.reshape(N_TRI // 128, 128))
        return out.reshape(N_ENT, N_ENT)
    return jnp.zeros((N_ENT, N_ENT), jnp.float32).at[h, t].add(scores)


def kernel(entity_emb_sr, entity_emb_tg, relation_emb_sr, relation_emb_tg,
           head_sr, tail_sr, relation_sr, head_tg, tail_tg, relation_tg):
    out_sr = _side(entity_emb_sr, relation_emb_sr, head_sr, tail_sr, relation_sr)
    out_tg = _side(entity_emb_tg, relation_emb_tg, head_tg, tail_tg, relation_tg)
    return (out_sr, out_tg)
